# baseline (speedup 1.0000x reference)
import functools

import jax
import jax.numpy as jnp
from jax.experimental import pallas as pl


def _final_body(v_ref, m_ref, sc_ref, sh_ref, o_ref):
    scale = sc_ref[0, :]
    shift = sh_ref[0, :]
    y = (v_ref[...] + m_ref[...]) * scale[None, :] + shift[None, :]
    o_ref[...] = jnp.where(y > 0, y, 0.2 * y)


def kernel(x, W, gamma, beta):
    B, C, N = x.shape
    k = 20
    O = W.shape[0]
    W1 = W[:, :C]
    W2 = W[:, C:]

    xt = jnp.transpose(x, (0, 2, 1))
    inner = -2.0 * jnp.matmul(xt, jnp.transpose(xt, (0, 2, 1)))
    xx = jnp.sum(xt ** 2, axis=2, keepdims=True)
    pd = -xx - inner - jnp.transpose(xx, (0, 2, 1))
    _, idx = jax.lax.top_k(pd, k)

    xt_flat = xt.reshape(B * N, C)
    u_t = xt_flat @ W1.T
    v_t = xt_flat @ (W2 - W1).T

    gidx = (idx + jnp.arange(B).reshape(B, 1, 1) * N).reshape(-1)
    g = u_t[gidx].reshape(B * N, k, O)
    m_t = jnp.max(g, axis=1)
    s_t = jnp.sum(g, axis=1)
    q_t = jnp.sum(g * g, axis=1)

    cnt = B * N * k
    sum_s = jnp.sum(s_t, axis=0)
    sum_q = jnp.sum(q_t, axis=0)
    sum_sv = jnp.sum(s_t * v_t, axis=0)
    sum_v = jnp.sum(v_t, axis=0)
    sum_v2 = jnp.sum(v_t * v_t, axis=0)
    mean = (sum_s + k * sum_v) / cnt
    e2 = (sum_q + 2.0 * sum_sv + k * sum_v2) / cnt
    var = e2 - mean * mean
    scale = gamma / jnp.sqrt(var + 1e-5)
    shift = beta - mean * scale

    RB = 512
    y_t = pl.pallas_call(
        _final_body,
        grid=(B * N // RB,),
        in_specs=[
            pl.BlockSpec((RB, O), lambda i: (i, 0)),
            pl.BlockSpec((RB, O), lambda i: (i, 0)),
            pl.BlockSpec((1, O), lambda i: (0, 0)),
            pl.BlockSpec((1, O), lambda i: (0, 0)),
        ],
        out_specs=pl.BlockSpec((RB, O), lambda i: (i, 0)),
        out_shape=jax.ShapeDtypeStruct((B * N, O), jnp.float32),
    )(v_t, m_t, scale.reshape(1, O), shift.reshape(1, O))

    return jnp.transpose(y_t.reshape(B, N, O), (0, 2, 1))


# speedup vs baseline: 9.3684x; 9.3684x over previous
import functools

import jax
import jax.numpy as jnp
from jax import lax
from jax.experimental import pallas as pl
from jax.experimental.pallas import tpu as pltpu
from jax.experimental.pallas import tpu_sc as plsc

B, C, N, K, O = 8, 64, 2048, 20, 64
RB = 256
NB = N // RB
NW = 32
PTS = B * N // NW
CP = 32
NCHUNK = PTS // CP
IPC = CP * K
IROWS = IPC // 128


def _topk_body(x_ref, w1_ref, wd_ref, idx_ref, u_ref, v_ref, xx_s):
    b = pl.program_id(0)
    r = pl.program_id(1)
    xb = x_ref[0]

    @pl.when(r == 0)
    def _():
        xx_s[...] = jnp.sum(xb * xb, axis=0, keepdims=True)

    xr = x_ref[0, :, pl.ds(r * RB, RB)]
    u = lax.dot_general(xr, w1_ref[...], (((0,), (0,)), ((), ())),
                        preferred_element_type=jnp.float32)
    u_ref[...] = jnp.concatenate([u, u * u], axis=1)
    v_ref[...] = lax.dot_general(xr, wd_ref[...], (((0,), (0,)), ((), ())),
                                 preferred_element_type=jnp.float32)
    g = lax.dot_general(xr, xb, (((0,), (0,)), ((), ())),
                        preferred_element_type=jnp.float32)
    xx = xx_s[...]
    xxr = xx_s[0, pl.ds(r * RB, RB)].reshape(RB, 1)
    pd = (2.0 * g - xxr) - xx
    iota = lax.broadcasted_iota(jnp.int32, (RB, N), 1)
    cols = []
    for _ in range(K):
        mx = jnp.max(pd, axis=1, keepdims=True)
        eq = pd == mx
        idxc = jnp.min(jnp.where(eq, iota, jnp.int32(N)), axis=1, keepdims=True)
        cols.append(idxc + b * N)
        pd = jnp.where(eq, jnp.float32(-3e38), pd)
    idx_ref[0] = jnp.concatenate(cols, axis=1)


def _gather_body(gidx_ref, u_ref, m_ref, s_ref, q_ref,
                 idx_v, rows_v, mb, sb, qb, sem):
    nc = 2
    wid = lax.axis_index("s") * nc + lax.axis_index("c")
    pltpu.sync_copy(gidx_ref.at[wid], idx_v)

    def chunk(c, carry):
        copies = [
            pltpu.async_copy(u_ref.at[idx_v.at[c * IROWS + g]],
                             rows_v.at[pl.ds(g * 128, 128)], sem)
            for g in range(IROWS)
        ]
        for cp in copies:
            cp.wait()

        def point(p, carry2):
            base = p * K
            for cg in range(4):
                sl = pl.ds(cg * 16, 16)
                sl2 = pl.ds(64 + cg * 16, 16)
                m = rows_v[base, sl]
                s = m
                q = rows_v[base, sl2]
                for j in range(1, K):
                    rj = rows_v[base + j, sl]
                    m = jnp.maximum(m, rj)
                    s = s + rj
                    q = q + rows_v[base + j, sl2]
                mb[p, sl] = m
                sb[p, sl] = s
                qb[p, sl] = q
            return carry2

        lax.fori_loop(0, CP, point, 0, unroll=False)
        obase = wid * PTS + c * CP
        pltpu.sync_copy(mb, m_ref.at[pl.ds(obase, CP)])
        pltpu.sync_copy(sb, s_ref.at[pl.ds(obase, CP)])
        pltpu.sync_copy(qb, q_ref.at[pl.ds(obase, CP)])
        return carry

    lax.fori_loop(0, NCHUNK, chunk, 0, unroll=False)


def _stats_body(s_ref, q_ref, v_ref, g_ref, b_ref, o_ref):
    cnt = jnp.float32(B * N * K)
    s = s_ref[...]
    q = q_ref[...]
    v = v_ref[...]
    sum_s = jnp.sum(s, axis=0)
    sum_q = jnp.sum(q, axis=0)
    sum_sv = jnp.sum(s * v, axis=0)
    sum_v = jnp.sum(v, axis=0)
    sum_v2 = jnp.sum(v * v, axis=0)
    mean = (sum_s + K * sum_v) / cnt
    e2 = (sum_q + 2.0 * sum_sv + K * sum_v2) / cnt
    var = e2 - mean * mean
    scale = g_ref[0, :] * lax.rsqrt(var + 1e-5)
    shift = b_ref[0, :] - mean * scale
    o_ref[0, :] = scale
    o_ref[1, :] = shift


def _final_body(v_ref, m_ref, ss_ref, o_ref):
    scale = ss_ref[0, :]
    shift = ss_ref[1, :]
    y = (v_ref[...] + m_ref[...]) * scale[None, :] + shift[None, :]
    o_ref[...] = jnp.where(y > 0, y, 0.2 * y)


@functools.partial(
    pl.kernel,
    mesh=plsc.VectorSubcoreMesh(core_axis_name="c", subcore_axis_name="s"),
    out_type=[
        jax.ShapeDtypeStruct((B * N, O), jnp.float32),
        jax.ShapeDtypeStruct((B * N, O), jnp.float32),
        jax.ShapeDtypeStruct((B * N, O), jnp.float32),
    ],
    scratch_types=[
        pltpu.VMEM((PTS * K // 128, 128), jnp.int32),
        pltpu.VMEM((IPC, 128), jnp.float32),
        pltpu.VMEM((CP, O), jnp.float32),
        pltpu.VMEM((CP, O), jnp.float32),
        pltpu.VMEM((CP, O), jnp.float32),
        pltpu.SemaphoreType.DMA,
    ],
)
def _sc_gather(gidx_ref, u_ref, m_ref, s_ref, q_ref,
               idx_v, rows_v, mb, sb, qb, sem):
    _gather_body(gidx_ref, u_ref, m_ref, s_ref, q_ref,
                 idx_v, rows_v, mb, sb, qb, sem)


def kernel(x, W, gamma, beta):
    w1t = W[:, :C].T
    wdt = (W[:, C:] - W[:, :C]).T

    gidx, u_t, v_t = pl.pallas_call(
        _topk_body,
        grid=(B, NB),
        in_specs=[
            pl.BlockSpec((1, C, N), lambda b, r: (b, 0, 0)),
            pl.BlockSpec((C, O), lambda b, r: (0, 0)),
            pl.BlockSpec((C, O), lambda b, r: (0, 0)),
        ],
        out_specs=[
            pl.BlockSpec((1, RB, K), lambda b, r: (b, r, 0)),
            pl.BlockSpec((RB, 2 * O), lambda b, r: (b * NB + r, 0)),
            pl.BlockSpec((RB, O), lambda b, r: (b * NB + r, 0)),
        ],
        out_shape=[
            jax.ShapeDtypeStruct((B, N, K), jnp.int32),
            jax.ShapeDtypeStruct((B * N, 2 * O), jnp.float32),
            jax.ShapeDtypeStruct((B * N, O), jnp.float32),
        ],
        scratch_shapes=[pltpu.VMEM((1, N), jnp.float32)],
    )(x, w1t, wdt)

    gidx3d = gidx.reshape(NW, PTS * K // 128, 128)
    m_t, s_t, q_t = _sc_gather(gidx3d, u_t)

    ss = pl.pallas_call(
        _stats_body,
        in_specs=[
            pl.BlockSpec((B * N, O), lambda: (0, 0)),
            pl.BlockSpec((B * N, O), lambda: (0, 0)),
            pl.BlockSpec((B * N, O), lambda: (0, 0)),
            pl.BlockSpec((1, O), lambda: (0, 0)),
            pl.BlockSpec((1, O), lambda: (0, 0)),
        ],
        out_specs=pl.BlockSpec((2, O), lambda: (0, 0)),
        out_shape=jax.ShapeDtypeStruct((2, O), jnp.float32),
    )(s_t, q_t, v_t, gamma.reshape(1, O), beta.reshape(1, O))

    FB = 2048
    y_t = pl.pallas_call(
        _final_body,
        grid=(B * N // FB,),
        in_specs=[
            pl.BlockSpec((FB, O), lambda i: (i, 0)),
            pl.BlockSpec((FB, O), lambda i: (i, 0)),
            pl.BlockSpec((2, O), lambda i: (0, 0)),
        ],
        out_specs=pl.BlockSpec((FB, O), lambda i: (i, 0)),
        out_shape=jax.ShapeDtypeStruct((B * N, O), jnp.float32),
    )(v_t, m_t, ss)

    return jnp.transpose(y_t.reshape(B, N, O), (0, 2, 1))


# speedup vs baseline: 9.9994x; 1.0673x over previous
import functools

import jax
import jax.numpy as jnp
from jax import lax
from jax.experimental import pallas as pl
from jax.experimental.pallas import tpu as pltpu
from jax.experimental.pallas import tpu_sc as plsc

B, C, N, K, O = 8, 64, 2048, 20, 64
RB = 256
NB = N // RB
NW = 32
GB = 2
NG = B // GB
PTS = GB * N // NW
CP = 32
NCHUNK = PTS // CP
IPC = CP * K
IROWS = IPC // 128


def _topk_body(x_ref, w1_ref, wd_ref, idx_ref, u_ref, v_ref, xx_s):
    b = pl.program_id(0)
    r = pl.program_id(1)
    xb = x_ref[0]

    @pl.when(r == 0)
    def _():
        xx_s[...] = jnp.sum(xb * xb, axis=0, keepdims=True)

    xr = x_ref[0, :, pl.ds(r * RB, RB)]
    u = lax.dot_general(xr, w1_ref[...], (((0,), (0,)), ((), ())),
                        preferred_element_type=jnp.float32)
    u_ref[...] = jnp.concatenate([u, u * u], axis=1)
    v_ref[...] = lax.dot_general(xr, wd_ref[...], (((0,), (0,)), ((), ())),
                                 preferred_element_type=jnp.float32)
    g = lax.dot_general(xr, xb, (((0,), (0,)), ((), ())),
                        preferred_element_type=jnp.float32)
    xx = xx_s[...]
    xxr = xx_s[0, pl.ds(r * RB, RB)].reshape(RB, 1)
    pd = (2.0 * g - xxr) - xx
    iota = lax.broadcasted_iota(jnp.int32, (RB, N), 1)
    cols = []
    for _ in range(K):
        mx = jnp.max(pd, axis=1, keepdims=True)
        eq = pd == mx
        idxc = jnp.min(jnp.where(eq, iota, jnp.int32(N)), axis=1, keepdims=True)
        cols.append(idxc + b * N)
        pd = jnp.where(eq, jnp.float32(-3e38), pd)
    idx_ref[0] = jnp.concatenate(cols, axis=1)


def _gather_body(gidx_ref, u_ref, m_ref, s_ref, q_ref,
                 idx_v, rows_v, mb, sb, qb, sem):
    nc = 2
    wid = lax.axis_index("s") * nc + lax.axis_index("c")
    pltpu.sync_copy(gidx_ref.at[wid], idx_v)

    def chunk(c, carry):
        copies = [
            pltpu.async_copy(u_ref.at[idx_v.at[c * IROWS + g]],
                             rows_v.at[pl.ds(g * 128, 128)], sem)
            for g in range(IROWS)
        ]
        for cp in copies:
            cp.wait()

        def point(p, carry2):
            base = p * K
            for cg in range(4):
                sl = pl.ds(cg * 16, 16)
                sl2 = pl.ds(64 + cg * 16, 16)
                m = rows_v[base, sl]
                s = m
                q = rows_v[base, sl2]
                for j in range(1, K):
                    rj = rows_v[base + j, sl]
                    m = jnp.maximum(m, rj)
                    s = s + rj
                    q = q + rows_v[base + j, sl2]
                mb[p, sl] = m
                sb[p, sl] = s
                qb[p, sl] = q
            return carry2

        lax.fori_loop(0, CP, point, 0, unroll=False)
        obase = wid * PTS + c * CP
        pltpu.sync_copy(mb, m_ref.at[pl.ds(obase, CP)])
        pltpu.sync_copy(sb, s_ref.at[pl.ds(obase, CP)])
        pltpu.sync_copy(qb, q_ref.at[pl.ds(obase, CP)])
        return carry

    lax.fori_loop(0, NCHUNK, chunk, 0, unroll=False)


def _stats_body(s_ref, q_ref, v_ref, g_ref, b_ref, o_ref):
    cnt = jnp.float32(B * N * K)
    s = s_ref[...]
    q = q_ref[...]
    v = v_ref[...]
    sum_s = jnp.sum(s, axis=0)
    sum_q = jnp.sum(q, axis=0)
    sum_sv = jnp.sum(s * v, axis=0)
    sum_v = jnp.sum(v, axis=0)
    sum_v2 = jnp.sum(v * v, axis=0)
    mean = (sum_s + K * sum_v) / cnt
    e2 = (sum_q + 2.0 * sum_sv + K * sum_v2) / cnt
    var = e2 - mean * mean
    scale = g_ref[0, :] * lax.rsqrt(var + 1e-5)
    shift = b_ref[0, :] - mean * scale
    o_ref[0, :] = scale
    o_ref[1, :] = shift


def _final_body(v_ref, m_ref, ss_ref, o_ref):
    scale = ss_ref[0, :]
    shift = ss_ref[1, :]
    y = (v_ref[...] + m_ref[...]) * scale[None, :] + shift[None, :]
    o_ref[...] = jnp.where(y > 0, y, 0.2 * y)


@functools.partial(
    pl.kernel,
    mesh=plsc.VectorSubcoreMesh(core_axis_name="c", subcore_axis_name="s"),
    out_type=[
        jax.ShapeDtypeStruct((GB * N, O), jnp.float32),
        jax.ShapeDtypeStruct((GB * N, O), jnp.float32),
        jax.ShapeDtypeStruct((GB * N, O), jnp.float32),
    ],
    scratch_types=[
        pltpu.VMEM((PTS * K // 128, 128), jnp.int32),
        pltpu.VMEM((IPC, 128), jnp.float32),
        pltpu.VMEM((CP, O), jnp.float32),
        pltpu.VMEM((CP, O), jnp.float32),
        pltpu.VMEM((CP, O), jnp.float32),
        pltpu.SemaphoreType.DMA,
    ],
)
def _sc_gather(gidx_ref, u_ref, m_ref, s_ref, q_ref,
               idx_v, rows_v, mb, sb, qb, sem):
    _gather_body(gidx_ref, u_ref, m_ref, s_ref, q_ref,
                 idx_v, rows_v, mb, sb, qb, sem)


def kernel(x, W, gamma, beta):
    w1t = W[:, :C].T
    wdt = (W[:, C:] - W[:, :C]).T

    ms, ss, qs, vs = [], [], [], []
    for g in range(NG):
        gidx, u_t, v_t = pl.pallas_call(
            _topk_body,
            grid=(GB, NB),
            in_specs=[
                pl.BlockSpec((1, C, N), lambda b, r: (b, 0, 0)),
                pl.BlockSpec((C, O), lambda b, r: (0, 0)),
                pl.BlockSpec((C, O), lambda b, r: (0, 0)),
            ],
            out_specs=[
                pl.BlockSpec((1, RB, K), lambda b, r: (b, r, 0)),
                pl.BlockSpec((RB, 2 * O), lambda b, r: (b * NB + r, 0)),
                pl.BlockSpec((RB, O), lambda b, r: (b * NB + r, 0)),
            ],
            out_shape=[
                jax.ShapeDtypeStruct((GB, N, K), jnp.int32),
                jax.ShapeDtypeStruct((GB * N, 2 * O), jnp.float32),
                jax.ShapeDtypeStruct((GB * N, O), jnp.float32),
            ],
            scratch_shapes=[pltpu.VMEM((1, N), jnp.float32)],
        )(lax.slice_in_dim(x, g * GB, (g + 1) * GB, axis=0), w1t, wdt)

        gidx3d = gidx.reshape(NW, PTS * K // 128, 128)
        m_g, s_g, q_g = _sc_gather(gidx3d, u_t)
        ms.append(m_g)
        ss.append(s_g)
        qs.append(q_g)
        vs.append(v_t)

    m_t = jnp.concatenate(ms, axis=0)
    s_t = jnp.concatenate(ss, axis=0)
    q_t = jnp.concatenate(qs, axis=0)
    v_t = jnp.concatenate(vs, axis=0)

    ss = pl.pallas_call(
        _stats_body,
        in_specs=[
            pl.BlockSpec((B * N, O), lambda: (0, 0)),
            pl.BlockSpec((B * N, O), lambda: (0, 0)),
            pl.BlockSpec((B * N, O), lambda: (0, 0)),
            pl.BlockSpec((1, O), lambda: (0, 0)),
            pl.BlockSpec((1, O), lambda: (0, 0)),
        ],
        out_specs=pl.BlockSpec((2, O), lambda: (0, 0)),
        out_shape=jax.ShapeDtypeStruct((2, O), jnp.float32),
    )(s_t, q_t, v_t, gamma.reshape(1, O), beta.reshape(1, O))

    FB = 2048
    y_t = pl.pallas_call(
        _final_body,
        grid=(B * N // FB,),
        in_specs=[
            pl.BlockSpec((FB, O), lambda i: (i, 0)),
            pl.BlockSpec((FB, O), lambda i: (i, 0)),
            pl.BlockSpec((2, O), lambda i: (0, 0)),
        ],
        out_specs=pl.BlockSpec((FB, O), lambda i: (i, 0)),
        out_shape=jax.ShapeDtypeStruct((B * N, O), jnp.float32),
    )(v_t, m_t, ss)

    return jnp.transpose(y_t.reshape(B, N, O), (0, 2, 1))
